# Initial kernel scaffold; baseline (speedup 1.0000x reference)
import jax
import jax.numpy as jnp
from jax.experimental import pallas as pl


def kernel(x, edge_index1, edge_index2, size1, size2, Wl1, Wr1, b1, Wl2, Wr2, b2):
    raise NotImplementedError("write your pallas kernel here")


# speedup vs baseline: 6.7704x; 6.7704x over previous
import functools

import jax
import jax.numpy as jnp
from jax import lax
from jax.experimental import pallas as pl
from jax.experimental.pallas import tpu as pltpu
from jax.experimental.pallas import tpu_sc as plsc

N, D = 10000, 128
T1, T2 = 4096, 1024
E1, E2 = 320000, 131072

NUM_CORES = 2
NUM_SUBCORES = 16
NW = NUM_CORES * NUM_SUBCORES


def _make_seg_sum(T, E, K):
  per_tile = E // NW
  assert per_tile * NW == E and per_tile % K == 0 and K % 16 == 0
  chunks = per_tile // K
  rpt = T // NUM_SUBCORES
  assert rpt % 16 == 0
  mesh = plsc.VectorSubcoreMesh(core_axis_name="c", subcore_axis_name="s")

  @functools.partial(
      pl.kernel,
      out_type=[
          jax.ShapeDtypeStruct((NUM_CORES, T, D), jnp.float32),
          jax.ShapeDtypeStruct((NUM_CORES, T), jnp.float32),
      ],
      mesh=mesh,
      scratch_types=[
          pltpu.VMEM((K,), jnp.int32),
          pltpu.VMEM((K,), jnp.int32),
          pltpu.VMEM((K, D), jnp.float32),
          pltpu.VMEM((K,), jnp.float32),
          pltpu.VMEM((16, D), jnp.float32),
          pltpu.VMEM((rpt,), jnp.float32),
          pltpu.VMEM_SHARED((T, D), jnp.float32),
          pltpu.VMEM_SHARED((T,), jnp.float32),
          pltpu.SemaphoreType.DMA,
      ],
  )
  def seg_sum(x_hbm, src_hbm, dst_hbm, acc_out, cnt_out,
              src_v, dst_v, rows_v, ones_v, zrow_v, zcnt_v,
              acc_sh, cnt_sh, sem):
    cid = lax.axis_index("c")
    sid = lax.axis_index("s")
    wid = sid * NUM_CORES + cid

    zero16 = jnp.zeros((16,), jnp.float32)
    one16 = jnp.ones((16,), jnp.float32)
    for r in range(16):
      for j in range(D // 16):
        zrow_v[r, pl.ds(j * 16, 16)] = zero16
    for j in range(K // 16):
      ones_v[pl.ds(j * 16, 16)] = one16
    for j in range(rpt // 16):
      zcnt_v[pl.ds(j * 16, 16)] = zero16

    row0 = pl.multiple_of(sid * rpt, 8)

    @pl.loop(0, rpt // 16)
    def _zero(t):
      pltpu.sync_copy(zrow_v, acc_sh.at[pl.ds(row0 + t * 16, 16)])

    pltpu.sync_copy(zcnt_v, cnt_sh.at[pl.ds(row0, rpt)])
    plsc.subcore_barrier()

    base0 = wid * per_tile

    @pl.loop(0, chunks)
    def _chunk(ci):
      base = pl.multiple_of(base0 + ci * K, 8)
      pltpu.sync_copy(src_hbm.at[pl.ds(base, K)], src_v)
      pltpu.sync_copy(dst_hbm.at[pl.ds(base, K)], dst_v)
      pltpu.async_copy(x_hbm.at[src_v], rows_v, sem).wait()
      pltpu.sync_copy(rows_v, acc_sh.at[dst_v], add=True)
      pltpu.sync_copy(ones_v, cnt_sh.at[dst_v], add=True)

    plsc.subcore_barrier()
    rsl = pl.ds(row0, rpt)
    pltpu.sync_copy(acc_sh.at[rsl], acc_out.at[cid, rsl])
    pltpu.sync_copy(cnt_sh.at[rsl], zcnt_v)
    pltpu.sync_copy(zcnt_v, cnt_out.at[cid, rsl])

  return seg_sum


_seg_sum_1 = _make_seg_sum(T1, E1, K=80)
_seg_sum_2 = _make_seg_sum(T2, E2, K=128)


def _make_linear(T, BR, last):
  grid = T // BR

  def body(acc0_ref, acc1_ref, cnt0_ref, cnt1_ref, x_ref, wl_ref, wr_ref,
           b_ref, o_ref):
    cnt = cnt0_ref[...] + cnt1_ref[...]
    agg = acc0_ref[...] + acc1_ref[...]
    mean = agg / jnp.maximum(cnt, 1.0)[:, None]
    z = (jnp.dot(mean, wl_ref[...], preferred_element_type=jnp.float32)
         + jnp.dot(x_ref[...], wr_ref[...], preferred_element_type=jnp.float32)
         + b_ref[...])
    if last:
      m = jnp.max(z, axis=-1, keepdims=True)
      e = jnp.exp(z - m)
      o_ref[...] = z - m - jnp.log(jnp.sum(e, axis=-1, keepdims=True))
    else:
      o_ref[...] = jnp.maximum(z, 0.0)

  return pl.pallas_call(
      body,
      grid=(grid,),
      in_specs=[
          pl.BlockSpec((BR, D), lambda i: (i, 0)),
          pl.BlockSpec((BR, D), lambda i: (i, 0)),
          pl.BlockSpec((BR,), lambda i: (i,)),
          pl.BlockSpec((BR,), lambda i: (i,)),
          pl.BlockSpec((BR, D), lambda i: (i, 0)),
          pl.BlockSpec((D, D), lambda i: (0, 0)),
          pl.BlockSpec((D, D), lambda i: (0, 0)),
          pl.BlockSpec((1, D), lambda i: (0, 0)),
      ],
      out_specs=pl.BlockSpec((BR, D), lambda i: (i, 0)),
      out_shape=jax.ShapeDtypeStruct((T, D), jnp.float32),
  )


_linear_1 = _make_linear(T1, 512, last=False)
_linear_2 = _make_linear(T2, 512, last=True)


def kernel(x, edge_index1, edge_index2, size1, size2,
           Wl1, Wr1, b1, Wl2, Wr2, b2):
  x4 = lax.dynamic_slice_in_dim(x, size1 - T1, T1, axis=0)
  acc_p, cnt_p = _seg_sum_1(x, edge_index1[0], edge_index1[1])
  h = _linear_1(acc_p[0], acc_p[1], cnt_p[0], cnt_p[1], x4,
                Wl1, Wr1, b1.reshape(1, D))
  h2 = lax.dynamic_slice_in_dim(h, size2 - T2, T2, axis=0)
  acc2_p, cnt2_p = _seg_sum_2(h, edge_index2[0], edge_index2[1])
  out = _linear_2(acc2_p[0], acc2_p[1], cnt2_p[0], cnt2_p[1], h2,
                  Wl2, Wr2, b2.reshape(1, D))
  return out
